# Initial kernel scaffold; baseline (speedup 1.0000x reference)
import jax
import jax.numpy as jnp
from jax.experimental import pallas as pl


def kernel(x, token_table, pos_table):
    raise NotImplementedError("write your pallas kernel here")


# speedup vs baseline: 1.1855x; 1.1855x over previous
import functools

import jax
import jax.numpy as jnp
from jax import lax
from jax.experimental import pallas as pl
from jax.experimental.pallas import tpu as pltpu
from jax.experimental.pallas import tpu_sc as plsc

VOCAB = 1000000
MAX_LEN = 200
EMBED = 32
BATCH = 4096
SEQ = 200

TOTAL = BATCH * SEQ
NC = 2
NS = 16
NW = NC * NS
PER_W = TOTAL // NW
CHUNK = 1024
N_CHUNK = PER_W // CHUNK
GATHER = 128
IDX_ROWS = CHUNK // GATHER


def _body(x_hbm, tok_hbm, pos_hbm, out_hbm, idx_v, rows_v, pos_v, sem):
    wid = lax.axis_index("s") * NC + lax.axis_index("c")
    pltpu.sync_copy(pos_hbm, pos_v)

    def chunk_body(g, carry):
        base = wid * PER_W + g * CHUNK
        xrow = wid * (PER_W // GATHER) + g * IDX_ROWS
        pltpu.sync_copy(x_hbm.at[pl.ds(xrow, IDX_ROWS)], idx_v)
        copies = [
            pltpu.async_copy(
                tok_hbm.at[idx_v.at[k]],
                rows_v.at[pl.ds(k * GATHER, GATHER)],
                sem,
            )
            for k in range(IDX_ROWS)
        ]
        for c in copies:
            c.wait()

        def row_body(i, c2):
            p = lax.rem(base + i, MAX_LEN)
            for h in range(2):
                sl = pl.ds(h * 16, 16)
                rows_v[i, sl] = rows_v[i, sl] + pos_v[p, sl]
            return c2

        lax.fori_loop(0, CHUNK, row_body, 0)
        pltpu.sync_copy(rows_v, out_hbm.at[pl.ds(base, CHUNK)])
        return carry

    lax.fori_loop(0, N_CHUNK, chunk_body, 0)


@functools.partial(
    pl.kernel,
    mesh=plsc.VectorSubcoreMesh(core_axis_name="c", subcore_axis_name="s"),
    out_type=jax.ShapeDtypeStruct((TOTAL, EMBED), jnp.float32),
    compiler_params=pltpu.CompilerParams(use_tc_tiling_on_sc=False),
    scratch_types=[
        pltpu.VMEM((IDX_ROWS, GATHER), jnp.int32),
        pltpu.VMEM((CHUNK, EMBED), jnp.float32),
        pltpu.VMEM((MAX_LEN, EMBED), jnp.float32),
        pltpu.SemaphoreType.DMA,
    ],
)
def _tok_pos_embed(x_hbm, tok_hbm, pos_hbm, out_hbm, idx_v, rows_v, pos_v, sem):
    _body(x_hbm, tok_hbm, pos_hbm, out_hbm, idx_v, rows_v, pos_v, sem)


def kernel(x, token_table, pos_table):
    x2d = x.astype(jnp.int32).reshape(TOTAL // GATHER, GATHER)
    out = _tok_pos_embed(x2d, token_table, pos_table)
    return out.reshape(BATCH, SEQ, EMBED)


# speedup vs baseline: 1.4980x; 1.2637x over previous
import functools

import jax
import jax.numpy as jnp
from jax import lax
from jax.experimental import pallas as pl
from jax.experimental.pallas import tpu as pltpu
from jax.experimental.pallas import tpu_sc as plsc

VOCAB = 1000000
MAX_LEN = 200
EMBED = 32
BATCH = 4096
SEQ = 200

NC = 2
NS = 16
NW = NC * NS
BT = BATCH // 128


def _gather(tok_hbm, idx_all_v, rows_v, sem, u):
    return pltpu.make_async_copy(
        tok_hbm.at[idx_all_v.at[u // 8, u % 8]], rows_v, sem
    )


def _out_copy(out_hbm, rows_v, osem, w, s):
    return pltpu.make_async_copy(
        rows_v, out_hbm.at[s, pl.ds(w * 128, 128)], osem
    )


def _body(x_hbm, tok_hbm, pos_hbm, out_hbm, idx_all_v, rows_a, rows_b, rows_c, rows_d, pos_v, sem, osem):
    w = lax.axis_index("s") * NC + lax.axis_index("c")
    copies = [
        pltpu.async_copy(x_hbm.at[st, w], idx_all_v.at[st], sem)
        for st in range(SEQ // 8)
    ]
    pltpu.sync_copy(pos_hbm, pos_v)
    for c in copies:
        c.wait()

    bufs = [rows_a, rows_b, rows_c, rows_d]
    _gather(tok_hbm, idx_all_v, rows_a, sem, 0).start()
    _gather(tok_hbm, idx_all_v, rows_b, sem, 1).start()

    def process(u, cur, nxt):
        _gather(tok_hbm, idx_all_v, cur, sem, u).wait()

        p0 = pos_v[u, pl.ds(0, 16)]
        p1 = pos_v[u, pl.ds(16, 16)]
        for j in range(128):
            cur[j, pl.ds(0, 16)] = cur[j, pl.ds(0, 16)] + p0
            cur[j, pl.ds(16, 16)] = cur[j, pl.ds(16, 16)] + p1

        _out_copy(out_hbm, cur, osem, w, u).start()

        @pl.when(u >= 2)
        def _():
            _out_copy(out_hbm, nxt, osem, w, u - 2).wait()

        @pl.when(u <= SEQ - 3)
        def _():
            _gather(tok_hbm, idx_all_v, nxt, sem, u + 2).start()

    def quad_body(k, carry):
        for i in range(4):
            u = 4 * k + i
            process(u, bufs[i], bufs[(i + 2) % 4])
        return carry

    lax.fori_loop(0, SEQ // 4, quad_body, 0)
    _out_copy(out_hbm, rows_c, osem, w, SEQ - 2).wait()
    _out_copy(out_hbm, rows_d, osem, w, SEQ - 1).wait()


@functools.partial(
    pl.kernel,
    mesh=plsc.VectorSubcoreMesh(core_axis_name="c", subcore_axis_name="s"),
    out_type=jax.ShapeDtypeStruct((SEQ, BATCH, EMBED), jnp.float32),
    compiler_params=pltpu.CompilerParams(use_tc_tiling_on_sc=False),
    scratch_types=[
        pltpu.VMEM((SEQ // 8, 8, 128), jnp.int32),
        pltpu.VMEM((128, EMBED), jnp.float32),
        pltpu.VMEM((128, EMBED), jnp.float32),
        pltpu.VMEM((128, EMBED), jnp.float32),
        pltpu.VMEM((128, EMBED), jnp.float32),
        pltpu.VMEM((MAX_LEN, EMBED), jnp.float32),
        pltpu.SemaphoreType.DMA,
        pltpu.SemaphoreType.DMA,
    ],
)
def _tok_pos_embed(x_hbm, tok_hbm, pos_hbm, out_hbm, idx_all_v, rows_a, rows_b, rows_c, rows_d, pos_v, sem, osem):
    _body(x_hbm, tok_hbm, pos_hbm, out_hbm, idx_all_v, rows_a, rows_b, rows_c, rows_d, pos_v, sem, osem)


def kernel(x, token_table, pos_table):
    x4d = (
        x.astype(jnp.int32)
        .T.reshape(SEQ // 8, 8, BT, 128)
        .transpose(0, 2, 1, 3)
    )
    out_sm = _tok_pos_embed(x4d, token_table, pos_table)
    return out_sm.transpose(1, 0, 2)
